# baseline (speedup 1.0000x reference)
import functools

import jax
import jax.numpy as jnp
from jax import lax
from jax.experimental import pallas as pl
from jax.experimental.pallas import tpu as pltpu
from jax.experimental.pallas import tpu_sc as plsc

D_MODEL = 768
VOCAB = 100000
ROWS = 8192
LANES = 16
DCHUNKS = D_MODEL // LANES

NC, NS = 2, 16
NW = NC * NS
RPW = ROWS // NW
R = 64
NCHUNK = RPW // R


def _embed_sc(tab_flat, x_flat):
    mesh = plsc.VectorSubcoreMesh(core_axis_name="c", subcore_axis_name="s",
                                  num_cores=NC, num_subcores=NS)

    @functools.partial(
        pl.kernel,
        out_type=jax.ShapeDtypeStruct((ROWS * D_MODEL,), jnp.float32),
        mesh=mesh,
        scratch_types=[
            pltpu.VMEM((RPW + LANES,), jnp.int32),
            pltpu.VMEM((R * D_MODEL,), jnp.int32),
            pltpu.VMEM((R * D_MODEL,), jnp.float32),
            pltpu.SemaphoreType.DMA,
        ],
    )
    def k(tab_hbm, x_hbm, out_hbm, x_v, idx_v, dat_v, sem):
        wid = lax.axis_index("s") * NC + lax.axis_index("c")
        base = wid * RPW
        pltpu.sync_copy(x_hbm.at[pl.ds(base, RPW)], x_v.at[pl.ds(0, RPW)])
        lane_off = lax.iota(jnp.int32, LANES) * VOCAB

        for c in range(NCHUNK):
            def row_body(r, _, c=c):
                xs = x_v[pl.ds(c * R + r, LANES)]
                v = lane_off + xs[0]
                for i in range(DCHUNKS):
                    idx_v[pl.ds(r * D_MODEL + i * LANES, LANES)] = v
                    if i + 1 < DCHUNKS:
                        v = v + (LANES * VOCAB)
                return 0

            lax.fori_loop(0, R, row_body, 0)
            pltpu.async_copy(tab_hbm.at[idx_v], dat_v, sem).wait()
            pltpu.sync_copy(
                dat_v, out_hbm.at[pl.ds((base + c * R) * D_MODEL, R * D_MODEL)])

    return k(tab_flat, x_flat)


def kernel(x, W_E):
    x_flat = x.reshape(-1).astype(jnp.int32)
    tab_flat = W_E.reshape(-1)
    out = _embed_sc(tab_flat, x_flat)
    return out.reshape(x.shape[0], x.shape[1], D_MODEL)


# speedup vs baseline: 23.7390x; 23.7390x over previous
import functools

import jax
import jax.numpy as jnp
from jax import lax
from jax.experimental import pallas as pl
from jax.experimental.pallas import tpu as pltpu
from jax.experimental.pallas import tpu_sc as plsc

D_MODEL = 768
VOCAB = 100000
ROWS = 8192

NC, NS = 2, 16
NW = NC * NS
RPW = ROWS // NW
R = 64
NCHUNK = RPW // R


def _gather_rows(tab, xf):
    mesh = plsc.VectorSubcoreMesh(core_axis_name="c", subcore_axis_name="s",
                                  num_cores=NC, num_subcores=NS)

    @functools.partial(
        pl.kernel,
        out_type=jax.ShapeDtypeStruct((ROWS, D_MODEL), jnp.float32),
        mesh=mesh,
        scratch_types=[
            pltpu.VMEM((RPW,), jnp.int32),
            pltpu.VMEM((R, D_MODEL), jnp.float32),
            pltpu.VMEM((R, D_MODEL), jnp.float32),
            pltpu.SemaphoreType.DMA,
            pltpu.SemaphoreType.DMA,
        ],
    )
    def k(tab_hbm, x_hbm, out_hbm, x_v, buf0, buf1, gsem, ssem):
        wid = lax.axis_index("s") * NC + lax.axis_index("c")
        base = wid * RPW
        pltpu.sync_copy(x_hbm.at[pl.ds(base, RPW)], x_v)
        bufs = [buf0, buf1]

        def start_gather(c):
            return pltpu.async_copy(
                tab_hbm.at[x_v.at[pl.ds(c * R, R)]], bufs[c % 2], gsem)

        def start_scatter(c):
            return pltpu.async_copy(
                bufs[c % 2], out_hbm.at[pl.ds(base + c * R, R)], ssem)

        gathers = {0: start_gather(0)}
        scatters = {}
        for c in range(NCHUNK):
            gathers[c].wait()
            if c >= 1:
                scatters[c - 1].wait()
            scatters[c] = start_scatter(c)
            if c + 1 < NCHUNK:
                gathers[c + 1] = start_gather(c + 1)
        scatters[NCHUNK - 1].wait()

    return k(tab, xf)


def kernel(x, W_E):
    xf = x.reshape(-1).astype(jnp.int32)
    tab = W_E.T
    out = _gather_rows(tab, xf)
    return out.reshape(x.shape[0], x.shape[1], D_MODEL)


# speedup vs baseline: 24.5263x; 1.0332x over previous
import functools

import jax
import jax.numpy as jnp
from jax import lax
from jax.experimental import pallas as pl
from jax.experimental.pallas import tpu as pltpu
from jax.experimental.pallas import tpu_sc as plsc

D_MODEL = 768
VOCAB = 100000
ROWS = 8192

NC, NS = 2, 16
NW = NC * NS
RPW = ROWS // NW
R = 32
NCHUNK = RPW // R
NBUF = 4


def _gather_rows(tab, xf):
    mesh = plsc.VectorSubcoreMesh(core_axis_name="c", subcore_axis_name="s",
                                  num_cores=NC, num_subcores=NS)

    @functools.partial(
        pl.kernel,
        out_type=jax.ShapeDtypeStruct((ROWS, D_MODEL), jnp.float32),
        mesh=mesh,
        scratch_types=[
            pltpu.VMEM((RPW,), jnp.int32),
            [pltpu.VMEM((R, D_MODEL), jnp.float32) for _ in range(NBUF)],
            pltpu.SemaphoreType.DMA,
            pltpu.SemaphoreType.DMA,
        ],
    )
    def k(tab_hbm, x_hbm, out_hbm, x_v, bufs, gsem, ssem):
        wid = lax.axis_index("s") * NC + lax.axis_index("c")
        base = wid * RPW
        pltpu.sync_copy(x_hbm.at[pl.ds(base, RPW)], x_v)

        def start_gather(c):
            return pltpu.async_copy(
                tab_hbm.at[x_v.at[pl.ds(c * R, R)]], bufs[c % NBUF], gsem)

        def start_scatter(c):
            return pltpu.async_copy(
                bufs[c % NBUF], out_hbm.at[pl.ds(base + c * R, R)], ssem)

        gathers = {c: start_gather(c) for c in range(2)}
        scatters = {}
        for c in range(NCHUNK):
            gathers[c].wait()
            scatters[c] = start_scatter(c)
            if c + 2 < NCHUNK:
                if c - 2 >= 0:
                    scatters[c - 2].wait()
                gathers[c + 2] = start_gather(c + 2)
        scatters[NCHUNK - 2].wait()
        scatters[NCHUNK - 1].wait()

    return k(tab, xf)


def kernel(x, W_E):
    xf = x.reshape(-1).astype(jnp.int32)
    tab = W_E.T
    out = _gather_rows(tab, xf)
    return out.reshape(x.shape[0], x.shape[1], D_MODEL)
